# baseline (speedup 1.0000x reference)
import jax
import jax.numpy as jnp
from jax import lax
from jax.experimental import pallas as pl
from jax.experimental.pallas import tpu as pltpu
from jax.experimental.pallas import tpu_sc as plsc

_NUM_ITEM = 1000000
_EMB = 32
_NEG = 4
_OUT_D = 2 * _EMB + 1

_NC = 2
_NS = 16
_NW = _NC * _NS

_DMA_ROWS = 128
_CHUNK = 512
_DPC = _CHUNK // _DMA_ROWS
_NBUF = 3
_NRBUF = 2


def _build_sc_call(batch):
    pos_per_w = batch // _NW
    neg_per_w = pos_per_w * _NEG
    n_chunks = 1 + neg_per_w // _CHUNK
    uneg_per_chunk = _CHUNK // _NEG

    mesh = plsc.VectorSubcoreMesh(
        core_axis_name="c", subcore_axis_name="s",
        num_cores=_NC, num_subcores=_NS)

    def body(user_hbm, item_hbm, neg_hbm, ue_hbm, ie_hbm, out_hbm,
             idx_u, idx_i, *bufs):
        u_v = bufs[0:_NBUF]
        i_v = bufs[_NBUF:2 * _NBUF]
        r_v = bufs[2 * _NBUF:2 * _NBUF + _NRBUF]
        d_v = bufs[2 * _NBUF + _NRBUF:2 * _NBUF + 2 * _NRBUF]
        gsem = bufs[2 * _NBUF + 2 * _NRBUF:3 * _NBUF + 2 * _NRBUF]
        wsem = bufs[3 * _NBUF + 2 * _NRBUF:]

        wid = lax.axis_index("s") * _NC + lax.axis_index("c")
        pltpu.sync_copy(user_hbm.at[pl.ds(wid * pos_per_w, pos_per_w)],
                        idx_u)
        pltpu.sync_copy(item_hbm.at[pl.ds(wid * pos_per_w, pos_per_w)],
                        idx_i.at[pl.ds(0, pos_per_w)])
        pltpu.sync_copy(neg_hbm.at[pl.ds(wid * neg_per_w, neg_per_w)],
                        idx_i.at[pl.ds(pos_per_w, neg_per_w)])

        def fire(c):
            b = c % _NBUF
            cps = []
            if c == 0:
                for k in range(_DPC):
                    dst = pl.ds(k * _DMA_ROWS, _DMA_ROWS)
                    cps.append(pltpu.async_copy(
                        ue_hbm.at[idx_u.at[pl.ds(k * _DMA_ROWS, _DMA_ROWS)]],
                        r_v[0].at[dst], gsem[b]))
            else:
                cps.append(pltpu.async_copy(
                    ue_hbm.at[idx_u.at[pl.ds((c - 1) * uneg_per_chunk,
                                             uneg_per_chunk)]],
                    u_v[b], gsem[b]))
            for k in range(_DPC):
                off = c * _CHUNK + k * _DMA_ROWS
                cps.append(pltpu.async_copy(
                    ie_hbm.at[idx_i.at[pl.ds(off, _DMA_ROWS)]],
                    i_v[b].at[pl.ds(k * _DMA_ROWS, _DMA_ROWS)], gsem[b]))
            return cps

        lanes = lax.iota(jnp.int32, 16)
        zeros16 = jnp.zeros((16,), jnp.int32)

        gath = {0: fire(0), 1: fire(1)}
        writes = {}
        for c in range(n_chunks):
            b = c % _NBUF
            rb = c % _NRBUF
            for g in gath.pop(c):
                g.wait()
            if c - 1 >= 0:
                for w in writes.pop(c - 1):
                    w.wait()
            if c + 2 < n_chunks:
                gath[c + 2] = fire(c + 2)

            neg = c > 0
            ub = u_v[b] if neg else r_v[0]

            def grp(g, _, ub=ub, ib=i_v[b], db=d_v[rb], neg=neg):
                rows = g * 16 + lanes
                urows = rows // _NEG if neg else rows
                acc = jnp.zeros((16,), jnp.float32)
                for cc in range(_EMB):
                    colv = jnp.full((16,), cc, jnp.int32)
                    acc = acc + (plsc.load_gather(ub, [urows, colv]) *
                                 plsc.load_gather(ib, [rows, colv]))
                plsc.store_scatter(db, [rows, zeros16], acc)
                return 0

            lax.fori_loop(0, _CHUNK // 16, grp, 0)

            if neg:
                def rep(j, _, ub=u_v[b], rbuf=r_v[rb]):
                    u0 = ub[j, pl.ds(0, 16)]
                    u1 = ub[j, pl.ds(16, 16)]
                    for t in range(_NEG):
                        rbuf[j * _NEG + t, pl.ds(0, 16)] = u0
                        rbuf[j * _NEG + t, pl.ds(16, 16)] = u1
                    return 0
                lax.fori_loop(0, uneg_per_chunk, rep, 0)
                base = batch + wid * neg_per_w + (c - 1) * _CHUNK
            else:
                base = wid * pos_per_w

            rows_sl = pl.ds(base, _CHUNK)
            writes[c] = [
                pltpu.async_copy(
                    r_v[rb], out_hbm.at[rows_sl, pl.ds(0, _EMB)], wsem[rb]),
                pltpu.async_copy(
                    i_v[b], out_hbm.at[rows_sl, pl.ds(_EMB, _EMB)], wsem[rb]),
                pltpu.async_copy(
                    d_v[rb], out_hbm.at[rows_sl, pl.ds(2 * _EMB, 1)],
                    wsem[rb]),
            ]
        for c in sorted(writes):
            for w in writes[c]:
                w.wait()

    scratch = (
        [pltpu.VMEM((uneg_per_chunk, _EMB), jnp.float32)
         for _ in range(_NBUF)] +
        [pltpu.VMEM((_CHUNK, _EMB), jnp.float32) for _ in range(_NBUF)] +
        [pltpu.VMEM((_CHUNK, _EMB), jnp.float32) for _ in range(_NRBUF)] +
        [pltpu.VMEM((_CHUNK, 1), jnp.float32) for _ in range(_NRBUF)] +
        [pltpu.SemaphoreType.DMA for _ in range(_NBUF + _NRBUF)]
    )

    return pl.kernel(
        body,
        out_type=jax.ShapeDtypeStruct((batch * (1 + _NEG), _OUT_D),
                                      jnp.float32),
        mesh=mesh,
        compiler_params=pltpu.CompilerParams(
            needs_layout_passes=False, use_tc_tiling_on_sc=False),
        scratch_types=[
            pltpu.VMEM((pos_per_w,), jnp.int32),
            pltpu.VMEM((pos_per_w + neg_per_w,), jnp.int32),
        ] + scratch,
    )


def kernel(user, item, user_emb, item_emb):
    B = user.shape[0]
    neg_item = jax.random.randint(
        jax.random.key(42), (B * _NEG,), 0, _NUM_ITEM, dtype=jnp.int32)
    call = _build_sc_call(B)
    return call(user, item, neg_item, user_emb, item_emb)


# speedup vs baseline: 1.0041x; 1.0041x over previous
import jax
import jax.numpy as jnp
from jax import lax
from jax.experimental import pallas as pl
from jax.experimental.pallas import tpu as pltpu
from jax.experimental.pallas import tpu_sc as plsc

_NUM_ITEM = 1000000
_EMB = 32
_NEG = 4
_OUT_D = 2 * _EMB + 1

_NC = 2
_NS = 16
_NW = _NC * _NS

_DMA_ROWS = 256
_CHUNK = 512
_DPC = _CHUNK // _DMA_ROWS
_NBUF = 3


def _build_sc_call(total_rows):
    rows_per_w = total_rows // _NW
    n_chunks = rows_per_w // _CHUNK

    mesh = plsc.VectorSubcoreMesh(
        core_axis_name="c", subcore_axis_name="s",
        num_cores=_NC, num_subcores=_NS)

    def body(nu_hbm, ni_hbm, ue_hbm, ie_hbm, out_hbm, idx_u, idx_i, *bufs):
        u_v = bufs[0:_NBUF]
        i_v = bufs[_NBUF:2 * _NBUF]
        d_v = bufs[2 * _NBUF:3 * _NBUF]
        gsem = bufs[3 * _NBUF:4 * _NBUF]
        wsem = bufs[4 * _NBUF:5 * _NBUF]

        wid = lax.axis_index("s") * _NC + lax.axis_index("c")
        pltpu.sync_copy(nu_hbm.at[pl.ds(wid * rows_per_w, rows_per_w)], idx_u)
        pltpu.sync_copy(ni_hbm.at[pl.ds(wid * rows_per_w, rows_per_w)], idx_i)

        def fire(c):
            b = c % _NBUF
            cps = []
            for k in range(_DPC):
                off = c * _CHUNK + k * _DMA_ROWS
                dst = pl.ds(k * _DMA_ROWS, _DMA_ROWS)
                cps.append(pltpu.async_copy(
                    ue_hbm.at[idx_u.at[pl.ds(off, _DMA_ROWS)]],
                    u_v[b].at[dst], gsem[b]))
                cps.append(pltpu.async_copy(
                    ie_hbm.at[idx_i.at[pl.ds(off, _DMA_ROWS)]],
                    i_v[b].at[dst], gsem[b]))
            return cps

        lanes = lax.iota(jnp.int32, 16)
        zeros16 = jnp.zeros((16,), jnp.int32)

        gath = {0: fire(0), 1: fire(1)}
        writes = {}
        for c in range(n_chunks):
            b = c % _NBUF
            for g in gath.pop(c):
                g.wait()
            if c + 2 < n_chunks:
                if c - 1 >= 0:
                    for w in writes.pop(c - 1):
                        w.wait()
                gath[c + 2] = fire(c + 2)

            def grp(g, _, ub=u_v[b], ib=i_v[b], db=d_v[b]):
                rows = g * 16 + lanes
                acc = jnp.zeros((16,), jnp.float32)
                for cc in range(_EMB):
                    colv = jnp.full((16,), cc, jnp.int32)
                    acc = acc + (plsc.load_gather(ub, [rows, colv]) *
                                 plsc.load_gather(ib, [rows, colv]))
                plsc.store_scatter(db, [rows, zeros16], acc)
                return 0

            lax.fori_loop(0, _CHUNK // 16, grp, 0)

            base = wid * rows_per_w + c * _CHUNK
            rows_sl = pl.ds(base, _CHUNK)
            writes[c] = [
                pltpu.async_copy(
                    u_v[b], out_hbm.at[rows_sl, pl.ds(0, _EMB)], wsem[b]),
                pltpu.async_copy(
                    i_v[b], out_hbm.at[rows_sl, pl.ds(_EMB, _EMB)], wsem[b]),
                pltpu.async_copy(
                    d_v[b], out_hbm.at[rows_sl, pl.ds(2 * _EMB, 1)], wsem[b]),
            ]
        for c in sorted(writes):
            for w in writes[c]:
                w.wait()

    scratch = (
        [pltpu.VMEM((_CHUNK, _EMB), jnp.float32) for _ in range(_NBUF)] +
        [pltpu.VMEM((_CHUNK, _EMB), jnp.float32) for _ in range(_NBUF)] +
        [pltpu.VMEM((_CHUNK, 1), jnp.float32) for _ in range(_NBUF)] +
        [pltpu.SemaphoreType.DMA for _ in range(2 * _NBUF)]
    )

    return pl.kernel(
        body,
        out_type=jax.ShapeDtypeStruct((total_rows, _OUT_D), jnp.float32),
        mesh=mesh,
        compiler_params=pltpu.CompilerParams(
            needs_layout_passes=False, use_tc_tiling_on_sc=False),
        scratch_types=[
            pltpu.VMEM((rows_per_w,), jnp.int32),
            pltpu.VMEM((rows_per_w,), jnp.int32),
        ] + scratch,
    )


def kernel(user, item, user_emb, item_emb):
    B = user.shape[0]
    total = B * (1 + _NEG)
    neg_item = jax.random.randint(
        jax.random.key(42), (B * _NEG,), 0, _NUM_ITEM, dtype=jnp.int32)
    new_user = jnp.concatenate([user, jnp.repeat(user, _NEG)], axis=0)
    new_item = jnp.concatenate([item, neg_item], axis=0)

    call = _build_sc_call(total)
    return call(new_user, new_item, user_emb, item_emb)


# speedup vs baseline: 1.0058x; 1.0017x over previous
import jax
import jax.numpy as jnp
from jax import lax
from jax.experimental import pallas as pl
from jax.experimental.pallas import tpu as pltpu
from jax.experimental.pallas import tpu_sc as plsc

_NUM_ITEM = 1000000
_EMB = 32
_NEG = 4
_OUT_D = 2 * _EMB + 1

_NC = 2
_NS = 16
_NW = _NC * _NS

_DMA_ROWS = 128
_CHUNK = 512
_DPC = _CHUNK // _DMA_ROWS
_NBUF = 3


def _build_sc_call(total_rows):
    rows_per_w = total_rows // _NW
    n_chunks = rows_per_w // _CHUNK

    mesh = plsc.VectorSubcoreMesh(
        core_axis_name="c", subcore_axis_name="s",
        num_cores=_NC, num_subcores=_NS)

    def body(nu_hbm, ni_hbm, ue_hbm, ie_hbm, out_hbm, idx_u, idx_i, *bufs):
        u_v = bufs[0:_NBUF]
        i_v = bufs[_NBUF:2 * _NBUF]
        d_v = bufs[2 * _NBUF:3 * _NBUF]
        gsem = bufs[3 * _NBUF:4 * _NBUF]
        wsem = bufs[4 * _NBUF:5 * _NBUF]

        wid = lax.axis_index("s") * _NC + lax.axis_index("c")
        pltpu.sync_copy(nu_hbm.at[pl.ds(wid * rows_per_w, rows_per_w)], idx_u)
        pltpu.sync_copy(ni_hbm.at[pl.ds(wid * rows_per_w, rows_per_w)], idx_i)

        def fire(c):
            b = c % _NBUF
            cps = []
            for k in range(_DPC):
                off = c * _CHUNK + k * _DMA_ROWS
                dst = pl.ds(k * _DMA_ROWS, _DMA_ROWS)
                cps.append(pltpu.async_copy(
                    ue_hbm.at[idx_u.at[pl.ds(off, _DMA_ROWS)]],
                    u_v[b].at[dst], gsem[b]))
                cps.append(pltpu.async_copy(
                    ie_hbm.at[idx_i.at[pl.ds(off, _DMA_ROWS)]],
                    i_v[b].at[dst], gsem[b]))
            return cps

        lanes = lax.iota(jnp.int32, 16)
        zeros16 = jnp.zeros((16,), jnp.int32)

        gath = {0: fire(0), 1: fire(1)}
        writes = {}
        for c in range(n_chunks):
            b = c % _NBUF
            for g in gath.pop(c):
                g.wait()
            if c + 2 < n_chunks:
                if c - 1 >= 0:
                    for w in writes.pop(c - 1):
                        w.wait()
                gath[c + 2] = fire(c + 2)

            def grp(g, _, ub=u_v[b], ib=i_v[b], db=d_v[b]):
                rows = g * 16 + lanes
                acc = jnp.zeros((16,), jnp.float32)
                for cc in range(_EMB):
                    colv = jnp.full((16,), cc, jnp.int32)
                    acc = acc + (plsc.load_gather(ub, [rows, colv]) *
                                 plsc.load_gather(ib, [rows, colv]))
                plsc.store_scatter(db, [rows, zeros16], acc)
                return 0

            lax.fori_loop(0, _CHUNK // 16, grp, 0)

            base = wid * rows_per_w + c * _CHUNK
            rows_sl = pl.ds(base, _CHUNK)
            writes[c] = [
                pltpu.async_copy(
                    u_v[b], out_hbm.at[rows_sl, pl.ds(0, _EMB)], wsem[b]),
                pltpu.async_copy(
                    i_v[b], out_hbm.at[rows_sl, pl.ds(_EMB, _EMB)], wsem[b]),
                pltpu.async_copy(
                    d_v[b], out_hbm.at[rows_sl, pl.ds(2 * _EMB, 1)], wsem[b]),
            ]
        for c in sorted(writes):
            for w in writes[c]:
                w.wait()

    scratch = (
        [pltpu.VMEM((_CHUNK, _EMB), jnp.float32) for _ in range(_NBUF)] +
        [pltpu.VMEM((_CHUNK, _EMB), jnp.float32) for _ in range(_NBUF)] +
        [pltpu.VMEM((_CHUNK, 1), jnp.float32) for _ in range(_NBUF)] +
        [pltpu.SemaphoreType.DMA for _ in range(2 * _NBUF)]
    )

    return pl.kernel(
        body,
        out_type=jax.ShapeDtypeStruct((total_rows, _OUT_D), jnp.float32),
        mesh=mesh,
        compiler_params=pltpu.CompilerParams(
            needs_layout_passes=False, use_tc_tiling_on_sc=False),
        scratch_types=[
            pltpu.VMEM((rows_per_w,), jnp.int32),
            pltpu.VMEM((rows_per_w,), jnp.int32),
        ] + scratch,
    )


def kernel(user, item, user_emb, item_emb):
    B = user.shape[0]
    total = B * (1 + _NEG)
    neg_item = jax.random.randint(
        jax.random.key(42), (B * _NEG,), 0, _NUM_ITEM, dtype=jnp.int32)
    new_user = jnp.concatenate([user, jnp.repeat(user, _NEG)], axis=0)
    new_item = jnp.concatenate([item, neg_item], axis=0)

    call = _build_sc_call(total)
    return call(new_user, new_item, user_emb, item_emb)
